# Initial kernel scaffold; baseline (speedup 1.0000x reference)
import jax
import jax.numpy as jnp
from jax.experimental import pallas as pl


def kernel(x, A_pos, W1, b1, W2, b2, W3, b3, Wg1, bg1, Wg2, bg2, Wg3, bg3, Wg4, bg4, Wg5, bg5, Wg6, bg6):
    raise NotImplementedError("write your pallas kernel here")


# speedup vs baseline: 3098.3514x; 3098.3514x over previous
import jax
import jax.numpy as jnp
from jax.experimental import pallas as pl

N = 1024


def _fused_kernel(at_ref, x_ref,
                  w1_ref, b1_ref, w2_ref, b2_ref, w3_ref, b3_ref,
                  wg1_ref, bg1_ref, wg2_ref, bg2_ref, wg3_ref, bg3_ref,
                  wg4_ref, bg4_ref, wg5_ref, bg5_ref, wg6_ref, bg6_ref,
                  o_ref):
    at = at_ref[...]
    row = jax.lax.broadcasted_iota(jnp.int32, (N, N), 0)
    col = jax.lax.broadcasted_iota(jnp.int32, (N, N), 1)
    aeff_t = jnp.where(row == col, 1.0, (at != 0).astype(jnp.float32))

    ones_col = jnp.ones((N, 1), jnp.float32)
    ones_row = jnp.ones((1, N), jnp.float32)
    dn = (((0,), (0,)), ((), ()))
    deg_col = jax.lax.dot_general(aeff_t, ones_col,
                                  (((1,), (0,)), ((), ())),
                                  preferred_element_type=jnp.float32)
    deg_row = jax.lax.dot_general(ones_row, aeff_t,
                                  (((1,), (1,)), ((), ())),
                                  preferred_element_type=jnp.float32)
    dinv_col = jnp.where(deg_col > 0, jax.lax.rsqrt(deg_col), 0.0)
    dinv_row = jnp.where(deg_row > 0, jax.lax.rsqrt(deg_row), 0.0)

    s_mat = aeff_t * dinv_col * dinv_row

    def mm(a, b):
        return jax.lax.dot_general(a, b, (((1,), (0,)), ((), ())),
                                   preferred_element_type=jnp.float32)

    def gcn(h, w, b):
        return mm(s_mat, mm(h, w)) + b

    relu = lambda v: jnp.maximum(v, 0.0)

    x = x_ref[...]
    x1l = mm(x, w1_ref[...]) + b1_ref[...]
    x1 = x1l + relu(gcn(x1l, wg1_ref[...], bg1_ref[...]))
    x2l = mm(x1, w2_ref[...]) + b2_ref[...]
    x2 = x2l + relu(gcn(x2l, wg2_ref[...], bg2_ref[...]))
    x3l = mm(x2, w3_ref[...]) + b3_ref[...]
    x3 = x3l + 0.5 * relu(gcn(x3l, wg3_ref[...], bg3_ref[...]))
    x4 = x3 + 0.5 * relu(gcn(x3, wg4_ref[...], bg4_ref[...]))
    x5 = x4 + 0.25 * relu(gcn(x4, wg5_ref[...], bg5_ref[...]))
    x6 = x5 + 0.25 * gcn(x5, wg6_ref[...], bg6_ref[...])
    o_ref[...] = x6


def kernel(x, A_pos, W1, b1, W2, b2, W3, b3, Wg1, bg1, Wg2, bg2, Wg3, bg3,
           Wg4, bg4, Wg5, bg5, Wg6, bg6):
    at = A_pos.T
    biases = [b.reshape(1, -1) for b in (b1, b2, b3, bg1, bg2, bg3, bg4, bg5, bg6)]
    b1r, b2r, b3r, bg1r, bg2r, bg3r, bg4r, bg5r, bg6r = biases
    out = pl.pallas_call(
        _fused_kernel,
        out_shape=jax.ShapeDtypeStruct((N, 128), jnp.float32),
    )(at, x, W1, b1r, W2, b2r, W3, b3r, Wg1, bg1r, Wg2, bg2r, Wg3, bg3r,
      Wg4, bg4r, Wg5, bg5r, Wg6, bg6r)
    return out


# speedup vs baseline: 4548.7858x; 1.4681x over previous
import jax
import jax.numpy as jnp
from jax.experimental import pallas as pl

N = 1024


def _fused_kernel(a_ref, x_ref,
                  w1_ref, b1_ref, w2_ref, b2_ref, w3_ref, b3_ref,
                  wg1_ref, bg1_ref, wg2_ref, bg2_ref, wg3_ref, bg3_ref,
                  wg4_ref, bg4_ref, wg5_ref, bg5_ref, wg6_ref, bg6_ref,
                  o_ref):
    a = a_ref[...]
    row = jax.lax.broadcasted_iota(jnp.int32, (N, N), 0)
    col = jax.lax.broadcasted_iota(jnp.int32, (N, N), 1)
    aeff = jnp.where((a != 0) | (row == col), 1.0, 0.0).astype(jnp.bfloat16)

    def mm_t(lhs, rhs):
        return jax.lax.dot_general(lhs, rhs, (((0,), (0,)), ((), ())),
                                   preferred_element_type=jnp.float32)

    def mm(lhs, rhs):
        return jax.lax.dot_general(lhs, rhs, (((1,), (0,)), ((), ())),
                                   preferred_element_type=jnp.float32)

    deg = mm_t(aeff, jnp.ones((N, 1), jnp.bfloat16))
    dinv = jnp.where(deg > 0, jax.lax.rsqrt(deg), 0.0)

    def gcn(h, w, b):
        q = (dinv * mm(h, w)).astype(jnp.bfloat16)
        return dinv * mm_t(aeff, q) + b

    relu = lambda v: jnp.maximum(v, 0.0)

    x = x_ref[...]
    x1l = mm(x, w1_ref[...]) + b1_ref[...]
    x1 = x1l + relu(gcn(x1l, wg1_ref[...], bg1_ref[...]))
    x2l = mm(x1, w2_ref[...]) + b2_ref[...]
    x2 = x2l + relu(gcn(x2l, wg2_ref[...], bg2_ref[...]))
    x3l = mm(x2, w3_ref[...]) + b3_ref[...]
    x3 = x3l + 0.5 * relu(gcn(x3l, wg3_ref[...], bg3_ref[...]))
    x4 = x3 + 0.5 * relu(gcn(x3, wg4_ref[...], bg4_ref[...]))
    x5 = x4 + 0.25 * relu(gcn(x4, wg5_ref[...], bg5_ref[...]))
    x6 = x5 + 0.25 * gcn(x5, wg6_ref[...], bg6_ref[...])
    o_ref[...] = x6


def kernel(x, A_pos, W1, b1, W2, b2, W3, b3, Wg1, bg1, Wg2, bg2, Wg3, bg3,
           Wg4, bg4, Wg5, bg5, Wg6, bg6):
    biases = [b.reshape(1, -1) for b in (b1, b2, b3, bg1, bg2, bg3, bg4, bg5, bg6)]
    b1r, b2r, b3r, bg1r, bg2r, bg3r, bg4r, bg5r, bg6r = biases
    out = pl.pallas_call(
        _fused_kernel,
        out_shape=jax.ShapeDtypeStruct((N, 128), jnp.float32),
    )(A_pos, x, W1, b1r, W2, b2r, W3, b3r, Wg1, bg1r, Wg2, bg2r, Wg3, bg3r,
      Wg4, bg4r, Wg5, bg5r, Wg6, bg6r)
    return out


# speedup vs baseline: 4553.9192x; 1.0011x over previous
import jax
import jax.numpy as jnp
from jax.experimental import pallas as pl

N = 1024


def _fused_kernel(a_ref, x_ref,
                  w1_ref, b1_ref, w2_ref, b2_ref, w3_ref, b3_ref,
                  wg1_ref, bg1_ref, wg2_ref, bg2_ref, wg3_ref, bg3_ref,
                  wg4_ref, bg4_ref, wg5_ref, bg5_ref, wg6_ref, bg6_ref,
                  o_ref):
    a = a_ref[...]
    row = jax.lax.broadcasted_iota(jnp.int32, (N, N), 0)
    col = jax.lax.broadcasted_iota(jnp.int32, (N, N), 1)
    aeff = jnp.where((a != 0) | (row == col), 1.0, 0.0).astype(jnp.bfloat16)

    def mm_t(lhs, rhs):
        return jax.lax.dot_general(lhs, rhs, (((0,), (0,)), ((), ())),
                                   preferred_element_type=jnp.float32)

    def mm(lhs, rhs):
        return jax.lax.dot_general(lhs, rhs, (((1,), (0,)), ((), ())),
                                   preferred_element_type=jnp.float32)

    bf = lambda v: v.astype(jnp.bfloat16)

    deg = mm_t(aeff, jnp.ones((N, 1), jnp.bfloat16))
    dinv = jnp.where(deg > 0, jax.lax.rsqrt(deg), 0.0)
    dinv_h, dinv_q = 0.5 * dinv, 0.25 * dinv

    def gcn(h, w_ref, b_ref, dout, bscale):
        q = bf(dinv * mm(h, w_ref[...]))
        return dout * mm_t(aeff, q) + bscale * b_ref[...]

    relu = lambda v: jnp.maximum(v, 0.0)

    x1l = mm(x_ref[...], w1_ref[...]) + b1_ref[...]
    x1 = x1l + relu(gcn(x1l, wg1_ref, bg1_ref, dinv, 1.0))
    x2l = mm(x1, w2_ref[...]) + b2_ref[...]
    x2 = x2l + relu(gcn(x2l, wg2_ref, bg2_ref, dinv, 1.0))
    x3l = mm(x2, w3_ref[...]) + b3_ref[...]
    x3 = x3l + relu(gcn(x3l, wg3_ref, bg3_ref, dinv_h, 0.5))
    x4 = x3 + relu(gcn(x3, wg4_ref, bg4_ref, dinv_h, 0.5))
    x5 = x4 + relu(gcn(x4, wg5_ref, bg5_ref, dinv_q, 0.25))
    x6 = x5 + gcn(x5, wg6_ref, bg6_ref, dinv_q, 0.25)
    o_ref[...] = x6


def kernel(x, A_pos, W1, b1, W2, b2, W3, b3, Wg1, bg1, Wg2, bg2, Wg3, bg3,
           Wg4, bg4, Wg5, bg5, Wg6, bg6):
    biases = [b.reshape(1, -1) for b in (b1, b2, b3, bg1, bg2, bg3, bg4, bg5, bg6)]
    b1r, b2r, b3r, bg1r, bg2r, bg3r, bg4r, bg5r, bg6r = biases
    out = pl.pallas_call(
        _fused_kernel,
        out_shape=jax.ShapeDtypeStruct((N, 128), jnp.float32),
    )(A_pos, x, W1, b1r, W2, b2r, W3, b3r, Wg1, bg1r, Wg2, bg2r, Wg3, bg3r,
      Wg4, bg4r, Wg5, bg5r, Wg6, bg6r)
    return out
